# baseline (speedup 1.0000x reference)
import jax
import jax.numpy as jnp
from jax.experimental import pallas as pl
from jax.experimental.pallas import tpu as pltpu

ORDER = 16
SUBSAMPLING_FACTOR = 2
NUM_FEATURES = 64
SUB_ORDER = ORDER // SUBSAMPLING_FACTOR


def _dma_kernel(in_hbm, out_hbm, sems):
    n_rows = out_hbm.shape[0]
    copies = []
    for i in range(n_rows):
        b, g = divmod(i, SUB_ORDER)
        src = b * ORDER + g * SUBSAMPLING_FACTOR
        copies.append(pltpu.make_async_copy(in_hbm.at[src], out_hbm.at[i], sems.at[i]))
    for c in copies:
        c.start()
    for c in copies:
        c.wait()


def kernel(x):
    B, C, H, W = x.shape
    row = NUM_FEATURES * H * W
    xr = x.reshape(B * ORDER, 512, row // 512)
    out = pl.pallas_call(
        _dma_kernel,
        in_specs=[pl.BlockSpec(memory_space=pltpu.MemorySpace.HBM)],
        out_specs=pl.BlockSpec(memory_space=pltpu.MemorySpace.HBM),
        out_shape=jax.ShapeDtypeStruct((B * SUB_ORDER, 512, row // 512), x.dtype),
        scratch_shapes=[pltpu.SemaphoreType.DMA((B * SUB_ORDER,))],
    )(xr)
    return out.reshape(B, SUB_ORDER * NUM_FEATURES, H, W)
